# baseline (speedup 1.0000x reference)
import functools

import jax
import jax.numpy as jnp
from jax import lax
from jax.experimental import pallas as pl
from jax.experimental.pallas import tpu as pltpu
from jax.experimental.pallas import tpu_sc as plsc

_N = 10000
_D = 128
_E = 320000
_NP = 10240
_KT = _NP // _D
_SLAB = _NP * _D

_NW = 32
_CH = 159
_C = _CH * 128
_TOT = _NW * _C

_sc_mesh = plsc.VectorSubcoreMesh(core_axis_name="c", subcore_axis_name="s")


@functools.partial(
    pl.kernel,
    out_type=(),
    mesh=_sc_mesh,
    scratch_types=[
        pltpu.VMEM((_C,), jnp.int32),
        pltpu.VMEM((_C,), jnp.int32),
        pltpu.VMEM((_C,), jnp.int32),
        pltpu.VMEM((_C,), jnp.float32),
        pltpu.SemaphoreType.DMA,
    ],
)
def _scatter_adj(src_hbm, dst_hbm, a_hbm, src_v, dst_v, idx_v, ones_v, sem):
    w = lax.axis_index("s") * 2 + lax.axis_index("c")
    pltpu.sync_copy(src_hbm.at[w], src_v)
    pltpu.sync_copy(dst_hbm.at[w], dst_v)

    def jbody(j, carry):
        sl = pl.ds(16 * j, 16)
        s = src_v[sl]
        d = dst_v[sl]
        idx_v[sl] = (d >> 7) * _SLAB + s * 128 + (d & 127)
        ones_v[sl] = jnp.ones((16,), jnp.float32)
        return carry

    lax.fori_loop(0, _C // 16, jbody, 0, unroll=8)

    pltpu.async_copy(ones_v, a_hbm.at[idx_v], sem).wait()


_BM = 512


def _make_spmm():
    def body(a_ref, x_ref, o_ref):
        k = pl.program_id(0)
        i = pl.program_id(1)
        sl = pl.ds(i * _BM, _BM)
        p = jnp.dot(a_ref[0], x_ref[0], preferred_element_type=jnp.float32)

        @pl.when(k == 0)
        def _():
            o_ref[sl, :] = p

        @pl.when(k != 0)
        def _():
            o_ref[sl, :] += p

    return pl.pallas_call(
        body,
        grid=(_KT, _NP // _BM),
        in_specs=[
            pl.BlockSpec((1, _BM, _D), lambda k, i: (k, i, 0)),
            pl.BlockSpec((1, _D, 2 * _D), lambda k, i: (k, 0, 0)),
        ],
        out_specs=pl.BlockSpec((_NP, 2 * _D), lambda k, i: (0, 0)),
        out_shape=jax.ShapeDtypeStruct((_NP, 2 * _D), jnp.float32),
    )


_spmm = _make_spmm()

_BE = 256


def _make_epilogue(relu, aug_out):
    def body(acc_ref, wt_ref, b_ref, o_ref):
        acc = acc_ref[...]
        deg = jnp.maximum(acc[:, _D:_D + 1], 0.5)
        y = jnp.dot(acc[:, :_D] / deg, wt_ref[...],
                    preferred_element_type=jnp.float32)
        y = y + b_ref[...]
        if relu:
            y = jnp.maximum(y, 0.0)
        if aug_out:
            col = lax.broadcasted_iota(jnp.int32, (_BE, _D), 1)
            ones = jnp.where(col == 0, 1.0, 0.0).astype(jnp.float32)
            o_ref[...] = jnp.concatenate([y, ones], axis=1).reshape(
                _BE // _D, _D, 2 * _D)
        else:
            o_ref[...] = y

    if aug_out:
        out_spec = pl.BlockSpec((_BE // _D, _D, 2 * _D), lambda i: (i, 0, 0))
        out_shape = jax.ShapeDtypeStruct((_KT, _D, 2 * _D), jnp.float32)
    else:
        out_spec = pl.BlockSpec((_BE, _D), lambda i: (i, 0))
        out_shape = jax.ShapeDtypeStruct((_NP, _D), jnp.float32)
    return pl.pallas_call(
        body,
        grid=(_NP // _BE,),
        in_specs=[
            pl.BlockSpec((_BE, 2 * _D), lambda i: (i, 0)),
            pl.BlockSpec((_D, _D), lambda i: (0, 0)),
            pl.BlockSpec((1, _D), lambda i: (0, 0)),
        ],
        out_specs=out_spec,
        out_shape=out_shape,
    )


_epi_mid = _make_epilogue(True, True)
_epi_fin = _make_epilogue(False, False)


def kernel(edges, graph_embedding, W1, b1, W2, b2):
    src = edges[:, 0]
    dst = edges[:, 1]
    ar = jnp.arange(_N, dtype=jnp.int32)
    pad = _TOT - (2 * _E + _N)
    zpad = jnp.zeros((pad,), jnp.int32)
    s_all = jnp.concatenate([src, dst, ar, zpad]).reshape(_NW, _C)
    d_all = jnp.concatenate([dst, src, ar, zpad]).reshape(_NW, _C)

    a_ref = jax.new_ref(jnp.zeros((_KT * _SLAB,), jnp.float32))
    _scatter_adj(s_all, d_all, a_ref)
    adj3 = a_ref[...].reshape(_KT, _NP, _D)

    x0 = jnp.zeros((_NP, _D), jnp.float32).at[:_N].set(graph_embedding)
    x3 = x0.reshape(_KT, _D, _D)
    extra = jnp.zeros((_KT, _D, _D), jnp.float32).at[:, :, 0].set(1.0)
    x0aug = jnp.concatenate([x3, extra], axis=2)

    acc1 = _spmm(adj3, x0aug)
    x1aug = _epi_mid(acc1, W1.T, b1.reshape(1, _D))
    acc2 = _spmm(adj3, x1aug)
    h2 = _epi_fin(acc2, W2.T, b2.reshape(1, _D))
    return h2[:_N]


# speedup vs baseline: 2.0637x; 2.0637x over previous
import functools

import jax
import jax.numpy as jnp
from jax import lax
from jax.experimental import pallas as pl
from jax.experimental.pallas import tpu as pltpu
from jax.experimental.pallas import tpu_sc as plsc

_N = 10000
_D = 128
_E = 320000
_NP = 10240
_KT = _NP // _D
_SLAB = _NP * _D

_NW = 32
_CH = 159
_C = _CH * 128
_TOT = _NW * _C

_sc_mesh = plsc.VectorSubcoreMesh(core_axis_name="c", subcore_axis_name="s")


@functools.partial(
    pl.kernel,
    out_type=(),
    mesh=_sc_mesh,
    scratch_types=[
        pltpu.VMEM((_C,), jnp.int32),
        pltpu.VMEM((_C,), jnp.int32),
        pltpu.VMEM((_C,), jnp.int32),
        pltpu.VMEM((_C,), jnp.float32),
        pltpu.SemaphoreType.DMA,
    ],
)
def _scatter_adj(src_hbm, dst_hbm, a_hbm, src_v, dst_v, idx_v, ones_v, sem):
    w = lax.axis_index("s") * 2 + lax.axis_index("c")
    pltpu.sync_copy(src_hbm.at[w], src_v)
    pltpu.sync_copy(dst_hbm.at[w], dst_v)

    def jbody(j, carry):
        sl = pl.ds(16 * j, 16)
        idx_v[sl] = src_v[sl] * _NP + dst_v[sl]
        ones_v[sl] = jnp.ones((16,), jnp.float32)
        return carry

    lax.fori_loop(0, _C // 16, jbody, 0, unroll=8)

    pltpu.async_copy(ones_v, a_hbm.at[idx_v], sem).wait()


_BM = 256


def _make_layer(relu):
    def body(a_ref, x_ref, wt_ref, b_ref, o_ref):
        accd = jnp.zeros((_BM, 2 * _D), jnp.float32)
        for k in range(_KT):
            accd += jnp.dot(a_ref[:, k, :], x_ref[k],
                            preferred_element_type=jnp.float32)
        acc = accd[:, :_D]
        deg = jnp.maximum(accd[:, _D:_D + 1], 0.5)
        y = jnp.dot(acc / deg, wt_ref[...], preferred_element_type=jnp.float32)
        y = y + b_ref[...]
        if relu:
            y = jnp.maximum(y, 0.0)
        o_ref[...] = y

    return pl.pallas_call(
        body,
        grid=(_NP // _BM,),
        in_specs=[
            pl.BlockSpec((_BM, _KT, _D), lambda i: (i, 0, 0)),
            pl.BlockSpec((_KT, _D, 2 * _D), lambda i: (0, 0, 0)),
            pl.BlockSpec((_D, _D), lambda i: (0, 0)),
            pl.BlockSpec((1, _D), lambda i: (0, 0)),
        ],
        out_specs=pl.BlockSpec((_BM, _D), lambda i: (i, 0)),
        out_shape=jax.ShapeDtypeStruct((_NP, _D), jnp.float32),
    )


_layer_relu = _make_layer(True)
_layer_lin = _make_layer(False)


def kernel(edges, graph_embedding, W1, b1, W2, b2):
    src = edges[:, 0]
    dst = edges[:, 1]
    ar = jnp.arange(_N, dtype=jnp.int32)
    pad = _TOT - (2 * _E + _N)
    zpad = jnp.zeros((pad,), jnp.int32)
    s_all = jnp.concatenate([src, dst, ar, zpad]).reshape(_NW, _C)
    d_all = jnp.concatenate([dst, src, ar, zpad]).reshape(_NW, _C)

    a_ref = jax.new_ref(jnp.zeros((_KT * _SLAB,), jnp.float32))
    _scatter_adj(s_all, d_all, a_ref)
    adj3 = a_ref[...].reshape(_NP, _KT, _D)

    x0 = jnp.zeros((_NP, _D), jnp.float32).at[:_N].set(graph_embedding)

    def aug(x):
        x3 = x.reshape(_KT, _D, _D)
        extra = jnp.zeros((_KT, _D, _D), jnp.float32).at[:, :, 0].set(1.0)
        return jnp.concatenate([x3, extra], axis=2)

    h1 = _layer_relu(adj3, aug(x0), W1.T, b1.reshape(1, _D))
    h2 = _layer_lin(adj3, aug(h1), W2.T, b2.reshape(1, _D))
    return h2[:_N]
